# baseline (speedup 1.0000x reference)
import functools

import jax
import jax.numpy as jnp
from jax import lax
from jax.experimental import pallas as pl
from jax.experimental.pallas import tpu as pltpu
from jax.experimental.pallas import tpu_sc as plsc

CH = 128
K = 4


@functools.lru_cache(maxsize=None)
def _make_lookup(n_b: int, n_l: int, vocab: int, emb: int):
    info = plsc.get_sparse_core_info()
    nc, ns = info.num_cores, info.num_subcores
    nw = nc * ns
    n_tok = n_b * n_l
    nch = n_tok // CH // nw
    groups = nch // K
    cpr = n_b // CH
    ntd = emb // 8
    assert n_tok % (CH * nw * K) == 0 and n_b % CH == 0 and emb % 8 == 0

    mesh = plsc.VectorSubcoreMesh(core_axis_name="c", subcore_axis_name="s")

    @functools.partial(
        pl.kernel,
        mesh=mesh,
        out_type=jax.ShapeDtypeStruct((n_l, ntd, cpr, 8 * CH), jnp.float32),
        compiler_params=pltpu.CompilerParams(
            use_tc_tiling_on_sc=False, needs_layout_passes=False),
        scratch_types=(
            [pltpu.VMEM((nch, CH), jnp.int32),
             pltpu.VMEM((K, CH, emb), jnp.float32),
             pltpu.VMEM((K, ntd, 8 * CH), jnp.float32)]
            + [pltpu.SemaphoreType.DMA] * (2 * K)
        ),
    )
    def lookup(ids_hbm, table_hbm, out_hbm, idx_v, rows_v, rows_t, *sems):
        sg, so = sems[:K], sems[K:]
        wid = lax.axis_index("s") * nc + lax.axis_index("c")
        cbase = wid * nch
        pltpu.sync_copy(ids_hbm.at[pl.ds(cbase, nch)], idx_v)

        iota16 = lax.iota(jnp.int32, 16)
        d0s = tuple(range(0, emb, 16))
        rvec = [(d0 + iota16) >> 3 for d0 in d0s]
        qvec = [((d0 + iota16) & 7) * CH for d0 in d0s]

        def fire_gather(j, b):
            return pltpu.async_copy(
                table_hbm.at[idx_v.at[j]], rows_v.at[b], sg[b])

        def wait_gather(j, b):
            pltpu.make_async_copy(
                table_hbm.at[idx_v.at[j]], rows_v.at[b], sg[b]).wait()

        def wait_out(j, b):
            gid = cbase + j
            pltpu.make_async_copy(
                rows_t.at[b], out_hbm.at[gid // cpr, :, gid % cpr], so[b]
            ).wait()

        UNROLL = 8

        def transpose_slot(b):
            def body(i0, carry):
                ibase = i0 * UNROLL
                for u in range(UNROLL):
                    i = ibase + u
                    bi = jnp.full((16,), 0, jnp.int32) + i
                    for t in range(len(d0s)):
                        val = rows_v[b, i, pl.ds(d0s[t], 16)]
                        plsc.store_scatter(
                            rows_t.at[b], [rvec[t], qvec[t] + bi], val)
                return carry
            lax.fori_loop(0, CH // UNROLL, body, 0)

        def fire_out(j, b):
            gid = cbase + j
            l = gid // cpr
            c0 = gid % cpr
            return pltpu.async_copy(
                rows_t.at[b], out_hbm.at[l, :, c0], so[b])

        def process(g, first):
            for b in range(K):
                j = g * K + b
                wait_gather(j, b)
                if not first:
                    wait_out(j - K, b)
                transpose_slot(b)
                fire_out(j, b)
                fire_gather(jnp.minimum(j + K, nch - 1), b)

        for b in range(K):
            fire_gather(b, b)
        process(0, True)

        def loop_body(g, carry):
            process(g, False)
            return carry
        lax.fori_loop(1, groups, loop_body, 0)

        for b in range(K):
            wait_gather(nch - 1, b)
            wait_out((groups - 1) * K + b, b)

    return lookup


def kernel(token_ids, embed_weight):
    n_b, n_l = token_ids.shape
    vocab, emb = embed_weight.shape
    ids2d = jnp.transpose(token_ids).reshape((n_b * n_l) // CH, CH)
    out4 = _make_lookup(n_b, n_l, vocab, emb)(ids2d, embed_weight)
    out5 = out4.reshape(n_l, emb // 8, n_b // CH, 8, CH)
    return out5.transpose(2, 4, 0, 1, 3).reshape(n_b, n_l, emb)


# speedup vs baseline: 1.2681x; 1.2681x over previous
import functools

import jax
import jax.numpy as jnp
from jax import lax
from jax.experimental import pallas as pl
from jax.experimental.pallas import tpu as pltpu
from jax.experimental.pallas import tpu_sc as plsc

CH = 128
K = 5
NBUF = 2 * K


@functools.lru_cache(maxsize=None)
def _make_lookup(n_tok: int, vocab: int, emb: int):
    info = plsc.get_sparse_core_info()
    nc, ns = info.num_cores, info.num_subcores
    nw = nc * ns
    nch = n_tok // CH // nw
    niter = nch // NBUF
    assert n_tok % (CH * nw) == 0 and nch % NBUF == 0

    mesh = plsc.VectorSubcoreMesh(core_axis_name="c", subcore_axis_name="s")

    @functools.partial(
        pl.kernel,
        mesh=mesh,
        out_type=jax.ShapeDtypeStruct((n_tok, emb), jnp.float32),
        compiler_params=pltpu.CompilerParams(
            use_tc_tiling_on_sc=False, needs_layout_passes=False),
        scratch_types=(
            [pltpu.VMEM((nch, CH), jnp.int32),
             pltpu.VMEM((NBUF, CH, emb), jnp.float32)]
            + [pltpu.SemaphoreType.DMA] * (2 * NBUF)
        ),
    )
    def lookup(ids_hbm, table_hbm, out_hbm, idx_v, rows_v, *sems):
        sg, so = sems[:NBUF], sems[NBUF:]
        wid = lax.axis_index("s") * nc + lax.axis_index("c")
        cbase = wid * nch
        pltpu.sync_copy(ids_hbm.at[pl.ds(cbase, nch)], idx_v)

        def fire_gather(j, b):
            pltpu.async_copy(table_hbm.at[idx_v.at[j]], rows_v.at[b], sg[b])

        def wait_gather(j, b):
            pltpu.make_async_copy(
                table_hbm.at[idx_v.at[j]], rows_v.at[b], sg[b]).wait()

        def out_slice(j):
            return out_hbm.at[pl.ds((cbase + j) * CH, CH)]

        def fire_out(j, b):
            pltpu.async_copy(rows_v.at[b], out_slice(j), so[b])

        def wait_out(j, b):
            pltpu.make_async_copy(rows_v.at[b], out_slice(j), so[b]).wait()

        def process(j0, first):
            for b in range(K):
                j = j0 + b
                wait_gather(j, b)
                fire_out(j, b)
                if not first:
                    wait_out(j - K, b + K)
                fire_gather(jnp.minimum(j + K, nch - 1), b + K)
            for b in range(K, NBUF):
                j = j0 + b
                wait_gather(j, b)
                fire_out(j, b)
                wait_out(j - K, b - K)
                fire_gather(jnp.minimum(j + K, nch - 1), b - K)

        for b in range(K):
            fire_gather(b, b)
        process(0, True)

        def loop_body(i, carry):
            process(i * NBUF, False)
            return carry
        lax.fori_loop(1, niter, loop_body, 0)

        for b in range(K):
            wait_gather(nch - 1, b)
        for b in range(K, NBUF):
            wait_out(nch - NBUF + b, b)

    return lookup


def kernel(token_ids, embed_weight):
    n_b, n_l = token_ids.shape
    vocab, emb = embed_weight.shape
    ids2d = token_ids.reshape((n_b * n_l) // CH, CH)
    out = _make_lookup(n_b * n_l, vocab, emb)(ids2d, embed_weight)
    return out.reshape(n_b, n_l, emb)


# speedup vs baseline: 1.5965x; 1.2589x over previous
import functools

import jax
import jax.numpy as jnp
from jax import lax
from jax.experimental import pallas as pl
from jax.experimental.pallas import tpu as pltpu
from jax.experimental.pallas import tpu_sc as plsc

CH = 128
NV = 8
NT = 4
LP = 133


@functools.lru_cache(maxsize=None)
def _make_lookup(n_b: int, n_l: int, vocab: int, emb: int):
    info = plsc.get_sparse_core_info()
    nc, ns = info.num_cores, info.num_subcores
    nw = nc * ns
    n_tok = n_b * n_l
    nch = n_tok // CH // nw
    niter = nch // NV
    cpr = n_b // CH
    ntr = emb // 8
    assert n_tok % (CH * nw) == 0 and nch % NV == 0 and n_b % CH == 0

    mesh = plsc.VectorSubcoreMesh(core_axis_name="c", subcore_axis_name="s")

    @functools.partial(
        pl.kernel,
        mesh=mesh,
        out_type=jax.ShapeDtypeStruct((n_l, ntr, cpr, 8, CH), jnp.float32),
        compiler_params=pltpu.CompilerParams(
            use_tc_tiling_on_sc=False, needs_layout_passes=False),
        scratch_types=(
            [pltpu.VMEM((nch, CH), jnp.int32),
             pltpu.VMEM((NV, CH, emb), jnp.float32),
             pltpu.VMEM((NT, ntr, 8, LP), jnp.float32)]
            + [pltpu.SemaphoreType.DMA] * (NV + NT)
        ),
    )
    def lookup(ids_hbm, table_hbm, out_hbm, idx_v, rows_v, rows_t, *sems):
        sg, so = sems[:NV], sems[NV:]
        wid = lax.axis_index("s") * nc + lax.axis_index("c")
        cbase = wid * nch
        pltpu.sync_copy(ids_hbm.at[pl.ds(cbase, nch)], idx_v)

        iota16 = lax.iota(jnp.int32, 16)
        trv = [2 * t + iota16 // 8 for t in range(emb // 16)]
        drv = iota16 % 8

        def fire_gather(j, b):
            pltpu.async_copy(table_hbm.at[idx_v.at[j]], rows_v.at[b], sg[b])

        def wait_gather(j, b):
            pltpu.make_async_copy(
                table_hbm.at[idx_v.at[j]], rows_v.at[b], sg[b]).wait()

        def out_block(j):
            gid = cbase + j
            return out_hbm.at[gid // cpr, :, gid % cpr]

        def src_block(b):
            return rows_t.at[b, :, :, pl.ds(0, CH)]

        def fire_out(j, b):
            pltpu.async_copy(src_block(b), out_block(j), so[b])

        def wait_out(j, b):
            pltpu.make_async_copy(src_block(b), out_block(j), so[b]).wait()

        UNROLL = 8

        def transpose(vb, tb):
            def body(i0, carry):
                ibase = i0 * UNROLL
                for u in range(UNROLL):
                    i = ibase + u
                    lane = jnp.full((16,), 0, jnp.int32) + i
                    for t in range(emb // 16):
                        val = rows_v[vb, i, pl.ds(16 * t, 16)]
                        plsc.store_scatter(
                            rows_t.at[tb], [trv[t], drv, lane], val)
                return carry
            lax.fori_loop(0, CH // UNROLL, body, 0)

        def process(j0, first):
            for b in range(NV):
                j = j0 + b
                tb = b % NT
                wait_gather(j, b)
                if not (first and b < NT):
                    wait_out(j - NT, tb)
                transpose(b, tb)
                fire_out(j, tb)
                fire_gather(jnp.minimum(j + NV, nch - 1), b)

        for b in range(NV):
            fire_gather(b, b)
        process(0, True)

        def loop_body(i, carry):
            process(i * NV, False)
            return carry
        lax.fori_loop(1, niter, loop_body, 0)

        for b in range(NV):
            wait_gather(nch - 1, b)
        for b in range(NT):
            wait_out(nch - NT + b, b)

    return lookup


def kernel(token_ids, embed_weight):
    n_b, n_l = token_ids.shape
    vocab, emb = embed_weight.shape
    ids2d = jnp.transpose(token_ids).reshape((n_b * n_l) // CH, CH)
    out5 = _make_lookup(n_b, n_l, vocab, emb)(ids2d, embed_weight)
    return out5.transpose(2, 4, 0, 1, 3).reshape(n_b, n_l, emb)
